# Initial kernel scaffold; baseline (speedup 1.0000x reference)
import jax
import jax.numpy as jnp
from jax.experimental import pallas as pl


def kernel(query, memory):
    raise NotImplementedError("write your pallas kernel here")


# speedup vs baseline: 28.4489x; 28.4489x over previous
import functools

import jax
import jax.numpy as jnp
from jax.experimental import pallas as pl
from jax.experimental.pallas import tpu as pltpu

K_FRAC = 0.01
_NEG_INF = float("-inf")


def _flip(j):
    return jnp.where(j < 0, j ^ jnp.int32(0x7FFFFFFF), j)


def _attn_kernel(q_ref, m_ref, out_ref, keys_ref, rowmax_ref,
                 thr_ref, z_ref, *, n, nt_total, k, qb, mt):
    ph = pl.program_id(1)
    nt = pl.program_id(2)

    @pl.when(ph == 0)
    def _phase0():
        s = jax.lax.dot_general(
            q_ref[...], m_ref[...], (((1,), (1,)), ((), ())),
            preferred_element_type=jnp.float32)
        col = jax.lax.broadcasted_iota(jnp.int32, (qb, mt), 1)
        valid = col < (n - nt * mt)
        key = jnp.where(valid, _flip(jax.lax.bitcast_convert_type(s, jnp.int32)),
                        jnp.iinfo(jnp.int32).min)
        keys_ref[nt] = key
        s_v = jnp.where(valid, s, jnp.float32(_NEG_INF))
        cur = jnp.max(s_v, axis=1, keepdims=True)
        prev = jnp.where(nt == 0, jnp.float32(_NEG_INF), rowmax_ref[...])
        rowmax_ref[...] = jnp.maximum(prev, cur)

    @pl.when(ph == 1)
    def _phase1():
        @pl.when(nt == 0)
        def _select():
            lo0 = jnp.full((qb, 1), jnp.iinfo(jnp.int32).min, jnp.int32)
            hi0 = jnp.full((qb, 1), jnp.iinfo(jnp.int32).max, jnp.int32)

            def count_ge(t):
                def body(i, c):
                    blk = keys_ref[i]
                    return c + jnp.sum((blk >= t).astype(jnp.int32), axis=1,
                                       keepdims=True)
                return jax.lax.fori_loop(
                    0, nt_total, body, jnp.zeros((qb, 1), jnp.int32))

            def step(_, carry):
                lo, hi = carry
                mid = (lo >> 1) + (hi >> 1) + (lo & hi & 1)
                c = count_ge(mid)
                ok = c >= k
                return jnp.where(ok, mid, lo), jnp.where(ok, hi, mid)

            lo, _ = jax.lax.fori_loop(0, 32, step, (lo0, hi0))
            thr_ref[...] = lo

        key_blk = keys_ref[nt]
        mask = key_blk >= thr_ref[...]
        s = jax.lax.bitcast_convert_type(_flip(key_blk), jnp.float32)
        w = jnp.where(mask, jnp.exp(s - rowmax_ref[...]), jnp.float32(0.0))
        part = jax.lax.dot_general(
            w, m_ref[...], (((1,), (0,)), ((), ())),
            preferred_element_type=jnp.float32)
        zcur = jnp.sum(w, axis=1, keepdims=True)
        prev_out = jnp.where(nt == 0, jnp.float32(0.0), out_ref[...])
        prev_z = jnp.where(nt == 0, jnp.float32(0.0), z_ref[...])
        out_ref[...] = prev_out + part
        z_ref[...] = prev_z + zcur

        @pl.when(nt == nt_total - 1)
        def _finish():
            out_ref[...] = out_ref[...] / z_ref[...]


def kernel(query, memory):
    b, qn, d = query.shape
    n, _ = memory.shape
    q2 = query.reshape(b * qn, d)
    nq = b * qn
    qb = 128 if nq % 128 == 0 else nq
    mt = 2048
    nt_total = -(-n // mt)
    n_pad = nt_total * mt
    k = int(n * K_FRAC)
    mem_p = jnp.pad(memory, ((0, n_pad - n), (0, 0)))

    grid = (nq // qb, 2, nt_total)

    out = pl.pallas_call(
        functools.partial(_attn_kernel, n=n, nt_total=nt_total, k=k,
                          qb=qb, mt=mt),
        grid=grid,
        in_specs=[
            pl.BlockSpec((qb, d), lambda g, p, t: (g, 0)),
            pl.BlockSpec((mt, d), lambda g, p, t: (t, 0)),
        ],
        out_specs=pl.BlockSpec((qb, d), lambda g, p, t: (g, 0)),
        out_shape=jax.ShapeDtypeStruct((nq, d), jnp.float32),
        scratch_shapes=[
            pltpu.VMEM((nt_total, qb, mt), jnp.int32),
            pltpu.VMEM((qb, 1), jnp.float32),
            pltpu.VMEM((qb, 1), jnp.int32),
            pltpu.VMEM((qb, 1), jnp.float32),
        ],
    )(q2, mem_p)
    return out.reshape(b, qn, d)
